# Initial kernel scaffold; baseline (speedup 1.0000x reference)
import jax
import jax.numpy as jnp
from jax.experimental import pallas as pl


def kernel(encoded, table, W, b):
    raise NotImplementedError("write your pallas kernel here")


# speedup vs baseline: 35.7668x; 35.7668x over previous
import functools

import jax
import jax.numpy as jnp
from jax import lax
from jax.experimental import pallas as pl
from jax.experimental.pallas import tpu as pltpu
from jax.experimental.pallas import tpu_sc as plsc

_D = 32
_L = 50
_B = 16384
_LANES = 16

_NC = 2
_NS = 16
_NW = _NC * _NS

_ROWS_PER_W = _B // _NW
_CB = 32
_NCH = _ROWS_PER_W // _CB
_CIDX = _CB * _L
_GSUB = 80
_NSUB = _CIDX // _GSUB


def _combine_rows(rows_v, w_v, b_v, out_v):
    def row_body(r, carry):
        base = r * _L
        acc0 = b_v[:]
        acc1 = b_v[:]
        for l in range(_L):
            wv = w_v[l, :]
            acc0 = acc0 + wv * rows_v[base + l, 0:_LANES]
            acc1 = acc1 + wv * rows_v[base + l, _LANES:_D]
        out_v[r, 0:_LANES] = jnp.maximum(acc0, 0.0)
        out_v[r, _LANES:_D] = jnp.maximum(acc1, 0.0)
        return carry

    lax.fori_loop(0, _CB, row_body, 0)


def _sc_body(enc_hbm, tab_hbm, w_hbm, b_hbm, out_hbm,
             idx0, idx1, rows0, rows1, out_v, w_v, b_v, sem0, sem1):
    wid = lax.axis_index("s") * _NC + lax.axis_index("c")
    row_base = wid * _ROWS_PER_W
    idx_base = row_base * _L

    pltpu.sync_copy(w_hbm, w_v)
    pltpu.sync_copy(b_hbm, b_v)

    bufs = ((idx0, rows0, sem0), (idx1, rows1, sem1))

    def start_chunk(c, idx_v, rows_v, sem):
        pltpu.sync_copy(enc_hbm.at[pl.ds(idx_base + c * _CIDX, _CIDX)], idx_v)
        for s in range(_NSUB):
            pltpu.async_copy(
                tab_hbm.at[idx_v.at[pl.ds(s * _GSUB, _GSUB)]],
                rows_v.at[pl.ds(s * _GSUB, _GSUB)],
                sem)

    def wait_chunk(rows_v, sem):
        pltpu.make_async_copy(tab_hbm.at[pl.ds(0, _CIDX)], rows_v, sem).wait()

    start_chunk(0, *bufs[0])
    start_chunk(1, *bufs[1])

    def outer(i, carry):
        for bsel in range(2):
            c = 2 * i + bsel
            idx_v, rows_v, sem = bufs[bsel]
            wait_chunk(rows_v, sem)
            _combine_rows(rows_v, w_v, b_v, out_v)
            pltpu.sync_copy(out_v, out_hbm.at[pl.ds(row_base + c * _CB, _CB)])

            @pl.when(c + 2 < _NCH)
            def _():
                start_chunk(c + 2, idx_v, rows_v, sem)

        return carry

    lax.fori_loop(0, _NCH // 2, outer, 0)


@jax.jit
def _run(enc_flat, table, w16, b16):
    mesh = plsc.VectorSubcoreMesh(core_axis_name="c", subcore_axis_name="s")
    sc = functools.partial(
        pl.kernel,
        out_type=jax.ShapeDtypeStruct((_B, _D), jnp.float32),
        mesh=mesh,
        scratch_types=[
            pltpu.VMEM((_CIDX,), jnp.int32),
            pltpu.VMEM((_CIDX,), jnp.int32),
            pltpu.VMEM((_CIDX, _D), jnp.float32),
            pltpu.VMEM((_CIDX, _D), jnp.float32),
            pltpu.VMEM((_CB, _D), jnp.float32),
            pltpu.VMEM((_L, _LANES), jnp.float32),
            pltpu.VMEM((_LANES,), jnp.float32),
            pltpu.SemaphoreType.DMA,
            pltpu.SemaphoreType.DMA,
        ],
        compiler_params=pltpu.CompilerParams(use_tc_tiling_on_sc=False),
    )(_sc_body)
    return sc(enc_flat, table, w16, b16)


def kernel(encoded, table, W, b):
    enc_flat = encoded.reshape(-1).astype(jnp.int32)
    w16 = jnp.broadcast_to(
        W.astype(jnp.float32).reshape(_L, 1), (_L, _LANES))
    b16 = jnp.broadcast_to(b.astype(jnp.float32).reshape(1), (_LANES,))
    return _run(enc_flat, table, w16, b16)
